# baseline (speedup 1.0000x reference)
import functools

import jax
import jax.numpy as jnp
from jax import lax
from jax.experimental import pallas as pl
from jax.experimental.pallas import tpu as pltpu
from jax.experimental.pallas import tpu_sc as plsc


def _table_transpose(V, D):
    VB = 1920
    grid = (V + VB - 1) // VB

    def body(i_ref, o_ref):
        x = i_ref[...].T
        o_ref[...] = jnp.concatenate([x, x], axis=1)

    return pl.pallas_call(
        body,
        grid=(grid,),
        in_specs=[pl.BlockSpec((D, VB), lambda i: (0, i))],
        out_specs=pl.BlockSpec((VB, 2 * D), lambda i: (i, 0)),
        out_shape=jax.ShapeDtypeStruct((V, 2 * D), jnp.float32),
    )


def _sc_gather(B, T, V, D):
    info = plsc.get_sparse_core_info()
    NC, NS, L = info.num_cores, info.num_subcores, info.num_lanes
    NW = NC * NS
    total = B * T
    per_w = total // NW
    CH = 800
    GI = 100
    NG = CH // GI
    n_chunks = per_w // CH

    assert total % (NW * CH) == 0 and GI <= 128
    assert D % L == 0 and n_chunks % 2 == 0

    mesh = plsc.VectorSubcoreMesh(core_axis_name="c", subcore_axis_name="s")

    @functools.partial(
        pl.kernel,
        mesh=mesh,
        out_type=jax.ShapeDtypeStruct((total, D), jnp.float32),
        scratch_types=[
            [pltpu.VMEM((NG, GI), jnp.int32)] * 2,
            [pltpu.VMEM((CH, D), jnp.float32)] * 2,
            [pltpu.SemaphoreType.DMA] * 2,
            [pltpu.SemaphoreType.DMA] * 2,
        ],
        compiler_params=pltpu.CompilerParams(use_tc_tiling_on_sc=False),
    )
    def sc_gather(idx_hbm, tok_hbm, out_hbm, idx_v, rows_v, gsem, ssem):
        wid = lax.axis_index("s") * NC + lax.axis_index("c")
        base = wid * per_w

        def chunk_off(k):
            return pl.multiple_of(base + k * CH, CH)

        def issue_chunk(k, b):
            off = chunk_off(k)
            pltpu.sync_copy(
                idx_hbm.at[pl.ds(pl.multiple_of(off // GI, NG), NG)], idx_v[b])
            for g in range(NG):
                pltpu.async_copy(
                    tok_hbm.at[idx_v[b].at[g]],
                    rows_v[b].at[pl.ds(g * GI, GI)],
                    gsem[b],
                )

        def wait_gathers(b):
            for g in range(NG):
                pltpu.make_async_copy(
                    tok_hbm.at[idx_v[b].at[g]],
                    rows_v[b].at[pl.ds(g * GI, GI)],
                    gsem[b],
                ).wait()

        def drain_store(b):
            pltpu.make_async_copy(
                rows_v[b], out_hbm.at[pl.ds(chunk_off(0), CH)], ssem[b]
            ).wait()

        issue_chunk(0, 0)

        def outer_body(c, carry):
            for b in (0, 1):
                k = 2 * c + b
                wait_gathers(b)

                @pl.when(k + 1 < n_chunks)
                def _():
                    @pl.when(k >= 1)
                    def _():
                        drain_store(1 - b)
                    issue_chunk(k + 1, 1 - b)

                pltpu.async_copy(
                    rows_v[b], out_hbm.at[pl.ds(chunk_off(k), CH)], ssem[b])
            return carry

        lax.fori_loop(0, n_chunks // 2, outer_body, 0)
        drain_store(0)
        drain_store(1)

    return sc_gather


def _retile_add(B, T, D):
    BB = 128
    KB = T * D // 128

    def body(x_ref, p_ref, o_ref):
        x = x_ref[...].reshape(BB, KB, 128).reshape(BB, T * D)
        x = x.T.reshape(T, D, BB)
        o_ref[...] = x + p_ref[...][:, :, None]

    return pl.pallas_call(
        body,
        grid=(B // BB,),
        in_specs=[
            pl.BlockSpec((BB * KB, 128), lambda i: (i, 0)),
            pl.BlockSpec((T, D), lambda i: (0, 0)),
        ],
        out_specs=pl.BlockSpec((T, D, BB), lambda i: (0, 0, i)),
        out_shape=jax.ShapeDtypeStruct((T, D, B), jnp.float32),
    )


def kernel(inputs, token_table, pos_table):
    B, T = inputs.shape
    V, D = token_table.shape
    tok_lin = jnp.concatenate([token_table, token_table], axis=1).reshape(2 * V, D)
    idx2d = (inputs.astype(jnp.int32) * 2).reshape(-1, 100)
    flat = _sc_gather(B, T, V, D)(idx2d, tok_lin)
    flat2 = flat.reshape(B * T * D // 128, 128)
    out_t = _retile_add(B, T, D)(flat2, pos_table)

    return out_t.transpose(2, 0, 1)


# speedup vs baseline: 1.5882x; 1.5882x over previous
import functools

import jax
import jax.numpy as jnp
from jax import lax
from jax.experimental import pallas as pl
from jax.experimental.pallas import tpu as pltpu
from jax.experimental.pallas import tpu_sc as plsc


def _table_transpose(V, D):
    VB = 7680
    grid = (V + VB - 1) // VB

    def body(i_ref, o_ref):
        x = i_ref[...].T
        o_ref[...] = jnp.concatenate([x, x], axis=1)

    return pl.pallas_call(
        body,
        grid=(grid,),
        in_specs=[pl.BlockSpec((D, VB), lambda i: (0, i))],
        out_specs=pl.BlockSpec((VB, 2 * D), lambda i: (i, 0)),
        out_shape=jax.ShapeDtypeStruct((V, 2 * D), jnp.float32),
    )


def _sc_gather(B, T, V, D):
    info = plsc.get_sparse_core_info()
    NC, NS, L = info.num_cores, info.num_subcores, info.num_lanes
    NW = NC * NS
    total = B * T
    per_w = total // NW
    CH = 800
    GI = 100
    NG = CH // GI
    n_chunks = per_w // CH

    assert total % (NW * CH) == 0 and GI <= 128
    assert D % L == 0 and n_chunks % 2 == 0

    mesh = plsc.VectorSubcoreMesh(core_axis_name="c", subcore_axis_name="s")

    @functools.partial(
        pl.kernel,
        mesh=mesh,
        out_type=jax.ShapeDtypeStruct((total, D), jnp.float32),
        scratch_types=[
            [pltpu.VMEM((NG, GI), jnp.int32)] * 2,
            [pltpu.VMEM((CH, D), jnp.float32)] * 2,
            [pltpu.SemaphoreType.DMA] * 2,
            [pltpu.SemaphoreType.DMA] * 2,
        ],
        compiler_params=pltpu.CompilerParams(use_tc_tiling_on_sc=False),
    )
    def sc_gather(idx_hbm, tok_hbm, out_hbm, idx_v, rows_v, gsem, ssem):
        wid = lax.axis_index("s") * NC + lax.axis_index("c")
        base = wid * per_w

        def chunk_off(k):
            return pl.multiple_of(base + k * CH, CH)

        def issue_chunk(k, b):
            off = chunk_off(k)
            pltpu.sync_copy(
                idx_hbm.at[pl.ds(pl.multiple_of(off // GI, NG), NG)], idx_v[b])
            for g in range(NG):
                pltpu.async_copy(
                    tok_hbm.at[idx_v[b].at[g]],
                    rows_v[b].at[pl.ds(g * GI, GI)],
                    gsem[b],
                )

        def wait_gathers(b):
            for g in range(NG):
                pltpu.make_async_copy(
                    tok_hbm.at[idx_v[b].at[g]],
                    rows_v[b].at[pl.ds(g * GI, GI)],
                    gsem[b],
                ).wait()

        def drain_store(b):
            pltpu.make_async_copy(
                rows_v[b], out_hbm.at[pl.ds(chunk_off(0), CH)], ssem[b]
            ).wait()

        issue_chunk(0, 0)

        def outer_body(c, carry):
            for b in (0, 1):
                k = 2 * c + b
                wait_gathers(b)

                @pl.when(k + 1 < n_chunks)
                def _():
                    @pl.when(k >= 1)
                    def _():
                        drain_store(1 - b)
                    issue_chunk(k + 1, 1 - b)

                pltpu.async_copy(
                    rows_v[b], out_hbm.at[pl.ds(chunk_off(k), CH)], ssem[b])
            return carry

        lax.fori_loop(0, n_chunks // 2, outer_body, 0)
        drain_store(0)
        drain_store(1)

    return sc_gather


def _retile_add(B, T, D, Bq, q, aliased):
    BB = 128
    KB = T * D // 128
    qb = q * (Bq // BB)

    def body(*refs):
        x_ref, p_ref, o_ref = refs[0], refs[1], refs[-1]
        x = x_ref[...].reshape(BB, KB, 128).reshape(BB, T * D)
        x = x.T.reshape(T, D, BB)
        o_ref[...] = x + p_ref[...][:, :, None]

    in_specs = [
        pl.BlockSpec((BB * KB, 128), lambda i: (i, 0)),
        pl.BlockSpec((T, D), lambda i: (0, 0)),
    ]
    aliases = {}
    if aliased:
        in_specs.append(pl.BlockSpec(memory_space=pl.ANY))
        aliases = {2: 0}

    return pl.pallas_call(
        body,
        grid=(Bq // BB,),
        in_specs=in_specs,
        out_specs=pl.BlockSpec((T, D, BB), lambda i: (0, 0, qb + i)),
        out_shape=jax.ShapeDtypeStruct((T, D, B), jnp.float32),
        input_output_aliases=aliases,
    )


def kernel(inputs, token_table, pos_table):
    B, T = inputs.shape
    V, D = token_table.shape
    NS = 4
    Bq = B // NS
    tok_lin = _table_transpose(V, D)(token_table.T).reshape(2 * V, D)
    idx2d = (inputs.astype(jnp.int32) * 2).reshape(NS, -1, 100)
    gather_q = _sc_gather(Bq, T, V, D)
    flats = [gather_q(idx2d[q], tok_lin).reshape(Bq * T * D // 128, 128)
             for q in range(NS)]
    out_t = _retile_add(B, T, D, Bq, 0, False)(flats[0], pos_table)
    for q in range(1, NS):
        out_t = _retile_add(B, T, D, Bq, q, True)(flats[q], pos_table, out_t)
    return out_t.transpose(2, 0, 1)


# speedup vs baseline: 1.6543x; 1.0416x over previous
import functools

import jax
import jax.numpy as jnp
from jax import lax
from jax.experimental import pallas as pl
from jax.experimental.pallas import tpu as pltpu
from jax.experimental.pallas import tpu_sc as plsc


def _table_transpose(V, D):
    VB = 15360
    grid = (V + VB - 1) // VB

    def body(i_ref, o_ref):
        x = i_ref[...].T
        o_ref[...] = jnp.concatenate([x, x], axis=1)

    return pl.pallas_call(
        body,
        grid=(grid,),
        in_specs=[pl.BlockSpec((D, VB), lambda i: (0, i))],
        out_specs=pl.BlockSpec((VB, 2 * D), lambda i: (i, 0)),
        out_shape=jax.ShapeDtypeStruct((V, 2 * D), jnp.float32),
    )


def _sc_gather(B, T, V, D):
    info = plsc.get_sparse_core_info()
    NC, NS, L = info.num_cores, info.num_subcores, info.num_lanes
    NW = NC * NS
    total = B * T
    per_w = total // NW
    CH = 800
    GI = 100
    NG = CH // GI
    n_chunks = per_w // CH

    assert total % (NW * CH) == 0 and GI <= 128
    assert D % L == 0 and n_chunks % 2 == 0

    mesh = plsc.VectorSubcoreMesh(core_axis_name="c", subcore_axis_name="s")

    @functools.partial(
        pl.kernel,
        mesh=mesh,
        out_type=jax.ShapeDtypeStruct((total, D), jnp.float32),
        scratch_types=[
            [pltpu.VMEM((NG, GI), jnp.int32)] * 2,
            [pltpu.VMEM((CH, D), jnp.float32)] * 2,
            [pltpu.SemaphoreType.DMA] * 2,
            [pltpu.SemaphoreType.DMA] * 2,
        ],
        compiler_params=pltpu.CompilerParams(use_tc_tiling_on_sc=False),
    )
    def sc_gather(idx_hbm, tok_hbm, out_hbm, idx_v, rows_v, gsem, ssem):
        wid = lax.axis_index("s") * NC + lax.axis_index("c")
        base = wid * per_w

        def chunk_off(k):
            return pl.multiple_of(base + k * CH, CH)

        def issue_chunk(k, b):
            off = chunk_off(k)
            pltpu.sync_copy(
                idx_hbm.at[pl.ds(pl.multiple_of(off // GI, NG), NG)], idx_v[b])
            for g in range(NG):
                pltpu.async_copy(
                    tok_hbm.at[idx_v[b].at[g]],
                    rows_v[b].at[pl.ds(g * GI, GI)],
                    gsem[b],
                )

        def wait_gathers(b):
            for g in range(NG):
                pltpu.make_async_copy(
                    tok_hbm.at[idx_v[b].at[g]],
                    rows_v[b].at[pl.ds(g * GI, GI)],
                    gsem[b],
                ).wait()

        def drain_store(b):
            pltpu.make_async_copy(
                rows_v[b], out_hbm.at[pl.ds(chunk_off(0), CH)], ssem[b]
            ).wait()

        issue_chunk(0, 0)

        def outer_body(c, carry):
            for b in (0, 1):
                k = 2 * c + b
                wait_gathers(b)

                @pl.when(k + 1 < n_chunks)
                def _():
                    @pl.when(k >= 1)
                    def _():
                        drain_store(1 - b)
                    issue_chunk(k + 1, 1 - b)

                pltpu.async_copy(
                    rows_v[b], out_hbm.at[pl.ds(chunk_off(k), CH)], ssem[b])
            return carry

        lax.fori_loop(0, n_chunks // 2, outer_body, 0)
        drain_store(0)
        drain_store(1)

    return sc_gather


def _retile_add(B, T, D, Bq, q, aliased):
    BB = 128
    KB = T * D // 128
    qb = q * (Bq // BB)

    def body(*refs):
        x_ref, p_ref, o_ref = refs[0], refs[1], refs[-1]
        x = x_ref[...].reshape(BB, KB, 128).reshape(BB, T * D)
        x = x.T.reshape(T, D, BB)
        o_ref[...] = x + p_ref[...][:, :, None]

    in_specs = [
        pl.BlockSpec((BB * KB, 128), lambda i: (i, 0)),
        pl.BlockSpec((T, D), lambda i: (0, 0)),
    ]
    aliases = {}
    if aliased:
        in_specs.append(pl.BlockSpec(memory_space=pl.ANY))
        aliases = {2: 0}

    return pl.pallas_call(
        body,
        grid=(Bq // BB,),
        in_specs=in_specs,
        out_specs=pl.BlockSpec((T, D, BB), lambda i: (0, 0, qb + i)),
        out_shape=jax.ShapeDtypeStruct((T, D, B), jnp.float32),
        input_output_aliases=aliases,
    )


def kernel(inputs, token_table, pos_table):
    B, T = inputs.shape
    V, D = token_table.shape
    NS = 8
    Bq = B // NS
    tok_lin = _table_transpose(V, D)(token_table.T).reshape(2 * V, D)
    idx2d = (inputs.astype(jnp.int32) * 2).reshape(NS, -1, 100)
    gather_q = _sc_gather(Bq, T, V, D)
    flats = [gather_q(idx2d[q], tok_lin).reshape(Bq * T * D // 128, 128)
             for q in range(NS)]
    out_t = _retile_add(B, T, D, Bq, 0, False)(flats[0], pos_table)
    for q in range(1, NS):
        out_t = _retile_add(B, T, D, Bq, q, True)(flats[q], pos_table, out_t)
    return out_t.transpose(2, 0, 1)
